# baseline (device time: 99876 ns/iter reference)
import jax
import jax.numpy as jnp
from jax import lax
from jax.experimental import pallas as pl
from jax.experimental.pallas import tpu as pltpu

N_DEV = 4
HQ = 8
DH = 128
SQ = 1024
SKV = 1024
WIN = 128
SCALE = 0.08838834764831843
CHUNK = SQ // N_DEV
KW = 512


def kernel(x, Wq, K_ext, V_ext, Wo):

    def body(x_ref, wq_ref, k_hbm, v_hbm, wo_ref, out_ref,
             k_ref, v_ref, wqb_ref, wob_ref, kb_ref, vb_ref,
             ctx_ref, acc_ref, rs_comm, ag_comm,
             rs_send, rs_recv, ag_send, ag_recv, copy_sems):
        my = lax.axis_index("i")
        left = lax.rem(my + N_DEV - 1, N_DEV)
        right = lax.rem(my + 1, N_DEV)
        diag = lax.rem(my + 2, N_DEV)

        h0 = my * HQ
        k_copy = pltpu.make_async_copy(
            k_hbm.at[0, :, pl.ds(h0, HQ), :], k_ref, copy_sems.at[0])
        v_copy = pltpu.make_async_copy(
            v_hbm.at[0, :, pl.ds(h0, HQ), :], v_ref, copy_sems.at[1])
        k_copy.start()
        v_copy.start()

        barrier_sem = pltpu.get_barrier_semaphore()
        for nbr in (left, right, diag):
            pl.semaphore_signal(
                barrier_sem, inc=1,
                device_id=(nbr,), device_id_type=pl.DeviceIdType.MESH)
        pl.semaphore_wait(barrier_sem, 3)

        wqb_ref[...] = wq_ref[...].astype(jnp.bfloat16)
        wob_ref[...] = wo_ref[...].astype(jnp.bfloat16)
        k_copy.wait()
        v_copy.wait()
        kb_ref[...] = k_ref[...].astype(jnp.bfloat16)
        vb_ref[...] = v_ref[...].astype(jnp.bfloat16)

        def compute_chunk(c: int):
            a = c * CHUNK
            start = min(max(a - WIN, 0), SKV - KW)
            qb = jnp.dot(x_ref[0, a:a + CHUNK, :].astype(jnp.bfloat16),
                         wqb_ref[...], preferred_element_type=jnp.float32)
            qi = a + lax.broadcasted_iota(jnp.int32, (CHUNK, KW), 0)
            ki = start + lax.broadcasted_iota(jnp.int32, (CHUNK, KW), 1)
            mask = jnp.abs(qi - ki) <= WIN
            for h in range(HQ):
                qh = qb[:, h * DH:(h + 1) * DH].astype(jnp.bfloat16)
                kh = kb_ref[start:start + KW, h, :]
                vh = vb_ref[start:start + KW, h, :]
                scores = lax.dot_general(
                    qh, kh, (((1,), (1,)), ((), ())),
                    preferred_element_type=jnp.float32) * SCALE
                scores = jnp.where(mask, scores, -1e9)
                m = jnp.max(scores, axis=-1, keepdims=True)
                e = jnp.exp(scores - m)
                s = jnp.sum(e, axis=-1, keepdims=True)
                w = (e / s).astype(jnp.bfloat16)
                ctx_ref[:, h * DH:(h + 1) * DH] = jnp.dot(
                    w, vh, preferred_element_type=jnp.float32)
            pout = jnp.dot(ctx_ref[...].astype(jnp.bfloat16), wob_ref[...],
                           preferred_element_type=jnp.float32)
            acc_ref[a:a + CHUNK, :] = pout.astype(jnp.bfloat16)

        def rdma_to(peer, comm, slot, send_sems, recv_sems, row_off):
            return pltpu.make_async_remote_copy(
                src_ref=acc_ref.at[pl.ds(row_off, CHUNK), :],
                dst_ref=comm.at[slot],
                send_sem=send_sems.at[slot],
                recv_sem=recv_sems.at[slot],
                device_id=(peer,), device_id_type=pl.DeviceIdType.MESH)

        am = my * CHUNK

        for c in range(N_DEV):
            compute_chunk(c)
            cc = jnp.int32(c)

            @pl.when(cc != my)
            def _():
                slot = lax.rem(cc - my + N_DEV, N_DEV) - 1
                rdma_to(cc, rs_comm, slot, rs_send, rs_recv,
                        c * CHUNK).start()

            @pl.when(cc == my)
            def _():
                for slot in range(N_DEV - 1):
                    rdma_to(my, rs_comm, slot, rs_send, rs_recv,
                            0).wait_recv()
                tot = (acc_ref[pl.ds(am, CHUNK), :].astype(jnp.float32)
                       + rs_comm[0].astype(jnp.float32)
                       + rs_comm[1].astype(jnp.float32)
                       + rs_comm[2].astype(jnp.float32))
                out_ref[0, pl.ds(am, CHUNK), :] = tot
                acc_ref[pl.ds(am, CHUNK), :] = tot.astype(jnp.bfloat16)
                for r in range(1, N_DEV):
                    peer = lax.rem(my + r, N_DEV)
                    rdma_to(peer, ag_comm, r - 1, ag_send, ag_recv,
                            am).start()

        for s in range(N_DEV - 1):
            rdma_to(my, ag_comm, s, ag_send, ag_recv, 0).wait_recv()
            owner = lax.rem(my - s - 1 + N_DEV, N_DEV)
            out_ref[0, pl.ds(owner * CHUNK, CHUNK), :] = ag_comm[
                s].astype(jnp.float32)

        for s in range(N_DEV - 1):
            rdma_to(my, rs_comm, s, rs_send, rs_recv, 0).wait_send()
            rdma_to(my, ag_comm, s, ag_send, ag_recv, 0).wait_send()

    return pl.pallas_call(
        body,
        out_shape=jax.ShapeDtypeStruct((1, SQ, SQ), jnp.float32),
        in_specs=[
            pl.BlockSpec(memory_space=pltpu.VMEM),
            pl.BlockSpec(memory_space=pltpu.VMEM),
            pl.BlockSpec(memory_space=pltpu.MemorySpace.HBM),
            pl.BlockSpec(memory_space=pltpu.MemorySpace.HBM),
            pl.BlockSpec(memory_space=pltpu.VMEM),
        ],
        out_specs=pl.BlockSpec(memory_space=pltpu.VMEM),
        scratch_shapes=[
            pltpu.VMEM((SKV, HQ, DH), jnp.float32),
            pltpu.VMEM((SKV, HQ, DH), jnp.float32),
            pltpu.VMEM((SQ, SQ), jnp.bfloat16),
            pltpu.VMEM((SQ, SQ), jnp.bfloat16),
            pltpu.VMEM((SKV, HQ, DH), jnp.bfloat16),
            pltpu.VMEM((SKV, HQ, DH), jnp.bfloat16),
            pltpu.VMEM((CHUNK, HQ * DH), jnp.float32),
            pltpu.VMEM((SQ, SQ), jnp.bfloat16),
            pltpu.VMEM((3, CHUNK, SQ), jnp.bfloat16),
            pltpu.VMEM((3, CHUNK, SQ), jnp.bfloat16),
            pltpu.SemaphoreType.DMA((3,)),
            pltpu.SemaphoreType.DMA((3,)),
            pltpu.SemaphoreType.DMA((3,)),
            pltpu.SemaphoreType.DMA((3,)),
            pltpu.SemaphoreType.DMA((2,)),
        ],
        compiler_params=pltpu.CompilerParams(collective_id=0),
    )(x, Wq, K_ext, V_ext, Wo)
